# Initial kernel scaffold; baseline (speedup 1.0000x reference)
import jax
import jax.numpy as jnp
from jax.experimental import pallas as pl


def kernel(x, edge_index, W1, b1, W2, b2):
    raise NotImplementedError("write your pallas kernel here")


# speedup vs baseline: 129.5346x; 129.5346x over previous
import functools

import jax
import jax.numpy as jnp
from jax import lax
from jax.experimental import pallas as pl
from jax.experimental.pallas import tpu as pltpu
from jax.experimental.pallas import tpu_sc as plsc

N_SC = 2
N_TILE = 16
NW = N_SC * N_TILE

N_NODES = 100000
NPAD = 102400
ROWS_PER_TILE = NPAD // N_TILE

N_EDGES = 6400000
EDGES_PER_TILE = N_EDGES // NW
WINDOW = 1600
N_ITER = EDGES_PER_TILE // WINDOW
STAGE_ITERS = ROWS_PER_TILE // WINDOW


def _sc_mesh():
    return plsc.VectorSubcoreMesh(core_axis_name="c", subcore_axis_name="s")


@functools.partial(
    pl.kernel,
    out_type=jax.ShapeDtypeStruct((N_SC * NPAD,), jnp.float32),
    mesh=_sc_mesh(),
    scratch_types=[
        pltpu.VMEM((WINDOW,), jnp.int32),
        pltpu.VMEM((WINDOW,), jnp.float32),
        pltpu.VMEM_SHARED((NPAD,), jnp.float32),
        pltpu.SemaphoreType.DMA,
    ],
    compiler_params=pltpu.CompilerParams(use_tc_tiling_on_sc=False),
)
def _deg_kernel(dst_hbm, ones_hbm, zeros_hbm, out_hbm, dst_v, ones_v,
                acc_sh, sem):
    cid = lax.axis_index("c")
    sid = lax.axis_index("s")
    wid = cid * N_TILE + sid
    r0 = sid * ROWS_PER_TILE
    for k in range(STAGE_ITERS):
        pltpu.sync_copy(zeros_hbm.at[pl.ds(r0 + k * WINDOW, WINDOW)], ones_v)
        pltpu.sync_copy(ones_v, acc_sh.at[pl.ds(r0 + k * WINDOW, WINDOW)])
    pltpu.sync_copy(ones_hbm, ones_v)
    plsc.subcore_barrier()

    def body(i, carry):
        base = wid * EDGES_PER_TILE + i * WINDOW
        pltpu.sync_copy(dst_hbm.at[pl.ds(base, WINDOW)], dst_v)
        pltpu.sync_copy(ones_v, acc_sh.at[dst_v], add=True)
        return carry

    lax.fori_loop(0, N_ITER, body, 0)
    plsc.subcore_barrier()
    for k in range(STAGE_ITERS):
        pltpu.sync_copy(acc_sh.at[pl.ds(r0 + k * WINDOW, WINDOW)], ones_v)
        pltpu.sync_copy(ones_v,
                        out_hbm.at[pl.ds(cid * NPAD + r0 + k * WINDOW, WINDOW)])


def _make_edge_kernel(feat):
    @functools.partial(
        pl.kernel,
        out_type=jax.ShapeDtypeStruct((N_SC, NPAD, feat), jnp.float32),
        mesh=_sc_mesh(),
        scratch_types=[
            pltpu.VMEM((WINDOW,), jnp.int32),
            pltpu.VMEM((WINDOW,), jnp.int32),
            pltpu.VMEM((WINDOW, feat), jnp.float32),
            pltpu.VMEM_SHARED((NPAD, feat), jnp.float32),
            pltpu.VMEM_SHARED((NPAD, feat), jnp.float32),
            pltpu.SemaphoreType.DMA,
        ],
        compiler_params=pltpu.CompilerParams(use_tc_tiling_on_sc=False),
    )
    def _edge_kernel(src_hbm, dst_hbm, z_hbm, zeros_hbm, out_hbm,
                     src_v, dst_v, msg_v, z_sh, acc_sh, sem):
        cid = lax.axis_index("c")
        sid = lax.axis_index("s")
        wid = cid * N_TILE + sid
        r0 = sid * ROWS_PER_TILE
        for k in range(STAGE_ITERS):
            rk = r0 + k * WINDOW
            pltpu.sync_copy(z_hbm.at[pl.ds(rk, WINDOW)], msg_v)
            pltpu.sync_copy(msg_v, z_sh.at[pl.ds(rk, WINDOW)])
            pltpu.sync_copy(zeros_hbm.at[pl.ds(rk, WINDOW)], msg_v)
            pltpu.sync_copy(msg_v, acc_sh.at[pl.ds(rk, WINDOW)])
        plsc.subcore_barrier()

        def body(i, carry):
            base = wid * EDGES_PER_TILE + i * WINDOW
            pltpu.sync_copy(src_hbm.at[pl.ds(base, WINDOW)], src_v)
            pltpu.sync_copy(dst_hbm.at[pl.ds(base, WINDOW)], dst_v)
            pltpu.async_copy(z_sh.at[src_v], msg_v, sem).wait()
            pltpu.sync_copy(msg_v, acc_sh.at[dst_v], add=True)
            return carry

        lax.fori_loop(0, N_ITER, body, 0)
        plsc.subcore_barrier()
        for k in range(STAGE_ITERS):
            rk = r0 + k * WINDOW
            pltpu.sync_copy(acc_sh.at[pl.ds(rk, WINDOW)], msg_v)
            pltpu.sync_copy(msg_v, out_hbm.at[cid, pl.ds(rk, WINDOW)])

    return _edge_kernel


_edge_kernel_4 = _make_edge_kernel(4)
_edge_kernel_8 = _make_edge_kernel(8)


def _tc1_body(degp_ref, xt_ref, w1t_ref, dis_ref, z1t_ref):
    deg = degp_ref[0] + degp_ref[1] + 1.0
    dis = lax.rsqrt(deg)[None, :]
    xwt = jnp.dot(w1t_ref[...], xt_ref[...],
                  preferred_element_type=jnp.float32)
    dis_ref[...] = dis
    z1t_ref[...] = dis * xwt


def _tc1(degp, xt, w1t):
    return pl.pallas_call(
        _tc1_body,
        out_shape=(
            jax.ShapeDtypeStruct((1, NPAD), jnp.float32),
            jax.ShapeDtypeStruct((4, NPAD), jnp.float32),
        ),
    )(degp, xt, w1t)


def _tc2_body(accpt_ref, z1t_ref, dis_ref, b1_ref, w2t_ref, z2t_ref):
    dis = dis_ref[...]
    agg = dis * (accpt_ref[0] + accpt_ref[1] + z1t_ref[...]) + b1_ref[...]
    h = jnp.maximum(agg, 0.0)
    hwt = jnp.dot(w2t_ref[...], h, preferred_element_type=jnp.float32)
    z2t_ref[...] = dis * hwt


def _tc2(accp1t, z1t, dis, b1c, w2t):
    return pl.pallas_call(
        _tc2_body,
        out_shape=jax.ShapeDtypeStruct((8, NPAD), jnp.float32),
    )(accp1t, z1t, dis, b1c, w2t)


def _tc3_body(accpt_ref, z2t_ref, dis_ref, b2_ref, out_ref):
    dis = dis_ref[...]
    out = dis * (accpt_ref[0] + accpt_ref[1] + z2t_ref[...]) + b2_ref[...]
    cols = lax.broadcasted_iota(jnp.int32, (8, NPAD), 1)
    valid = cols < N_NODES
    neg = jnp.full_like(out, -jnp.inf)
    masked = jnp.where(valid, out, neg)
    m = jnp.max(masked, axis=1, keepdims=True)
    s = jnp.sum(jnp.where(valid, jnp.exp(masked - m), 0.0), axis=1,
                keepdims=True)
    out_ref[...] = out - (m + jnp.log(s))


def _tc3(accp2t, z2t, dis, b2c):
    return pl.pallas_call(
        _tc3_body,
        out_shape=jax.ShapeDtypeStruct((8, NPAD), jnp.float32),
    )(accp2t, z2t, dis, b2c)


def kernel(x, edge_index, W1, b1, W2, b2):
    n = x.shape[0]
    src = edge_index[0].astype(jnp.int32)
    dst = edge_index[1].astype(jnp.int32)

    ones_w = jnp.ones((WINDOW,), jnp.float32)
    zeros1 = jnp.zeros((NPAD,), jnp.float32)
    zeros4 = jnp.zeros((NPAD, 4), jnp.float32)
    zeros8 = jnp.zeros((NPAD, 8), jnp.float32)
    xt = jnp.zeros((5, NPAD), jnp.float32).at[:, :n].set(x.T)

    degp = _deg_kernel(dst, ones_w, zeros1)
    dis, z1t = _tc1(degp.reshape(N_SC, NPAD), xt, W1.T)
    accp1 = _edge_kernel_4(src, dst, z1t.T, zeros4)
    z2t = _tc2(accp1.transpose(0, 2, 1), z1t, dis, b1.reshape(4, 1), W2.T)
    accp2 = _edge_kernel_8(src, dst, z2t.T, zeros8)
    outt = _tc3(accp2.transpose(0, 2, 1), z2t, dis, b2.reshape(8, 1))
    return outt.T[:n]


# speedup vs baseline: 165.8678x; 1.2805x over previous
import functools

import jax
import jax.numpy as jnp
from jax import lax
from jax.experimental import pallas as pl
from jax.experimental.pallas import tpu as pltpu
from jax.experimental.pallas import tpu_sc as plsc

N_SC = 2
N_TILE = 16
NW = N_SC * N_TILE

N_NODES = 100000
NPAD = 102400
ROWS_PER_TILE = NPAD // N_TILE

N_EDGES = 6400000
EDGES_PER_TILE = N_EDGES // NW

UNROLL = 5
WIN_E = 800
NIT_E = EDGES_PER_TILE // WIN_E
WIN_D = 1600
NIT_D = EDGES_PER_TILE // WIN_D


def _sc_mesh():
    return plsc.VectorSubcoreMesh(core_axis_name="c", subcore_axis_name="s")


def _stage_chunks(window):
    chunks = []
    off = 0
    while off < ROWS_PER_TILE:
        sz = min(window, ROWS_PER_TILE - off)
        chunks.append((off, sz))
        off += sz
    return chunks


@functools.partial(
    pl.kernel,
    out_type=jax.ShapeDtypeStruct((N_SC * NPAD,), jnp.float32),
    mesh=_sc_mesh(),
    scratch_types=[
        [pltpu.VMEM((WIN_D,), jnp.int32) for _ in range(UNROLL)],
        pltpu.VMEM((WIN_D,), jnp.float32),
        pltpu.VMEM_SHARED((NPAD,), jnp.float32),
        [pltpu.SemaphoreType.DMA for _ in range(UNROLL)],
        [pltpu.SemaphoreType.DMA for _ in range(2)],
    ],
)
def _deg_kernel(dst_hbm, ones_hbm, zeros_hbm, out_hbm,
                dst_v, ones_v, acc_sh, sidx, ssc):
    cid = lax.axis_index("c")
    sid = lax.axis_index("s")
    wid = cid * N_TILE + sid
    e0 = wid * EDGES_PER_TILE
    r0 = sid * ROWS_PER_TILE

    pltpu.sync_copy(zeros_hbm.at[pl.ds(0, WIN_D)], ones_v)
    for off, sz in _stage_chunks(WIN_D):
        pltpu.sync_copy(ones_v.at[pl.ds(0, sz)],
                        acc_sh.at[pl.ds(r0 + off, sz)])
    pltpu.sync_copy(ones_hbm, ones_v)
    plsc.subcore_barrier()

    def body(k, carry):
        idx_d = []
        for j in range(UNROLL):
            base = e0 + (k * UNROLL + j) * WIN_D
            idx_d.append(pltpu.async_copy(dst_hbm.at[pl.ds(base, WIN_D)],
                                          dst_v[j], sidx[j]))
        prev = [None, None]
        for j in range(UNROLL):
            idx_d[j].wait()
            if prev[j % 2] is not None:
                prev[j % 2].wait()
            prev[j % 2] = pltpu.async_copy(ones_v, acc_sh.at[dst_v[j]],
                                           ssc[j % 2], add=True)
        for p in prev:
            if p is not None:
                p.wait()
        return carry

    lax.fori_loop(0, NIT_D // UNROLL, body, 0)
    plsc.subcore_barrier()
    for off, sz in _stage_chunks(WIN_D):
        pltpu.sync_copy(acc_sh.at[pl.ds(r0 + off, sz)],
                        ones_v.at[pl.ds(0, sz)])
        pltpu.sync_copy(ones_v.at[pl.ds(0, sz)],
                        out_hbm.at[pl.ds(cid * NPAD + r0 + off, sz)])


def _make_edge_kernel(feat):
    @functools.partial(
        pl.kernel,
        out_type=jax.ShapeDtypeStruct((N_SC, NPAD, feat), jnp.float32),
        mesh=_sc_mesh(),
        scratch_types=[
            [pltpu.VMEM((WIN_E,), jnp.int32) for _ in range(UNROLL)],
            [pltpu.VMEM((WIN_E,), jnp.int32) for _ in range(UNROLL)],
            [pltpu.VMEM((WIN_E, feat), jnp.float32) for _ in range(2)],
            pltpu.VMEM_SHARED((NPAD, feat), jnp.float32),
            pltpu.VMEM_SHARED((NPAD, feat), jnp.float32),
            [pltpu.SemaphoreType.DMA for _ in range(UNROLL)],
            pltpu.SemaphoreType.DMA,
            pltpu.SemaphoreType.DMA,
        ],
        compiler_params=pltpu.CompilerParams(use_tc_tiling_on_sc=False),
    )
    def _edge_kernel(src_hbm, dst_hbm, z_hbm, zeros_hbm, out_hbm,
                     src_v, dst_v, msg_v, z_sh, acc_sh, sidx, sg, ssc):
        cid = lax.axis_index("c")
        sid = lax.axis_index("s")
        wid = cid * N_TILE + sid
        e0 = wid * EDGES_PER_TILE
        r0 = sid * ROWS_PER_TILE

        pltpu.sync_copy(zeros_hbm.at[pl.ds(0, WIN_E)], msg_v[0])
        for off, sz in _stage_chunks(WIN_E):
            pltpu.sync_copy(msg_v[0].at[pl.ds(0, sz)],
                            acc_sh.at[pl.ds(r0 + off, sz)])
        for off, sz in _stage_chunks(WIN_E):
            pltpu.sync_copy(z_hbm.at[pl.ds(r0 + off, sz)],
                            msg_v[1].at[pl.ds(0, sz)])
            pltpu.sync_copy(msg_v[1].at[pl.ds(0, sz)],
                            z_sh.at[pl.ds(r0 + off, sz)])
        plsc.subcore_barrier()

        def body(k, carry):
            idx_d = []
            for j in range(UNROLL):
                base = e0 + (k * UNROLL + j) * WIN_E
                idx_d.append((
                    pltpu.async_copy(src_hbm.at[pl.ds(base, WIN_E)],
                                     src_v[j], sidx[j]),
                    pltpu.async_copy(dst_hbm.at[pl.ds(base, WIN_E)],
                                     dst_v[j], sidx[j]),
                ))
            prev = None
            for j in range(UNROLL):
                idx_d[j][0].wait()
                idx_d[j][1].wait()
                pltpu.async_copy(z_sh.at[src_v[j]], msg_v[j % 2], sg).wait()
                if prev is not None:
                    prev.wait()
                prev = pltpu.async_copy(msg_v[j % 2], acc_sh.at[dst_v[j]],
                                        ssc, add=True)
            prev.wait()
            return carry

        lax.fori_loop(0, NIT_E // UNROLL, body, 0)
        plsc.subcore_barrier()
        for off, sz in _stage_chunks(WIN_E):
            pltpu.sync_copy(acc_sh.at[pl.ds(r0 + off, sz)],
                            msg_v[0].at[pl.ds(0, sz)])
            pltpu.sync_copy(msg_v[0].at[pl.ds(0, sz)],
                            out_hbm.at[cid, pl.ds(r0 + off, sz)])

    return _edge_kernel


_edge_kernel_4 = _make_edge_kernel(4)
_edge_kernel_8 = _make_edge_kernel(8)


def _tc1_body(degp_ref, xt_ref, w1t_ref, dis_ref, z1t_ref):
    deg = degp_ref[0] + degp_ref[1] + 1.0
    dis = lax.rsqrt(deg)[None, :]
    xwt = jnp.dot(w1t_ref[...], xt_ref[...],
                  preferred_element_type=jnp.float32)
    dis_ref[...] = dis
    z1t_ref[...] = dis * xwt


def _tc1(degp, xt, w1t):
    return pl.pallas_call(
        _tc1_body,
        out_shape=(
            jax.ShapeDtypeStruct((1, NPAD), jnp.float32),
            jax.ShapeDtypeStruct((4, NPAD), jnp.float32),
        ),
    )(degp, xt, w1t)


def _tc2_body(accpt_ref, z1t_ref, dis_ref, b1_ref, w2t_ref, z2t_ref):
    dis = dis_ref[...]
    agg = dis * (accpt_ref[0] + accpt_ref[1] + z1t_ref[...]) + b1_ref[...]
    h = jnp.maximum(agg, 0.0)
    hwt = jnp.dot(w2t_ref[...], h, preferred_element_type=jnp.float32)
    z2t_ref[...] = dis * hwt


def _tc2(accp1t, z1t, dis, b1c, w2t):
    return pl.pallas_call(
        _tc2_body,
        out_shape=jax.ShapeDtypeStruct((8, NPAD), jnp.float32),
    )(accp1t, z1t, dis, b1c, w2t)


def _tc3_body(accpt_ref, z2t_ref, dis_ref, b2_ref, out_ref):
    dis = dis_ref[...]
    out = dis * (accpt_ref[0] + accpt_ref[1] + z2t_ref[...]) + b2_ref[...]
    cols = lax.broadcasted_iota(jnp.int32, (8, NPAD), 1)
    valid = cols < N_NODES
    neg = jnp.full_like(out, -jnp.inf)
    masked = jnp.where(valid, out, neg)
    m = jnp.max(masked, axis=1, keepdims=True)
    s = jnp.sum(jnp.where(valid, jnp.exp(masked - m), 0.0), axis=1,
                keepdims=True)
    out_ref[...] = out - (m + jnp.log(s))


def _tc3(accp2t, z2t, dis, b2c):
    return pl.pallas_call(
        _tc3_body,
        out_shape=jax.ShapeDtypeStruct((8, NPAD), jnp.float32),
    )(accp2t, z2t, dis, b2c)


def kernel(x, edge_index, W1, b1, W2, b2):
    n = x.shape[0]
    src = edge_index[0].astype(jnp.int32)
    dst = edge_index[1].astype(jnp.int32)

    ones_w = jnp.ones((WIN_D,), jnp.float32)
    zeros1 = jnp.zeros((NPAD,), jnp.float32)
    zeros4 = jnp.zeros((NPAD, 4), jnp.float32)
    zeros8 = jnp.zeros((NPAD, 8), jnp.float32)
    xt = jnp.zeros((5, NPAD), jnp.float32).at[:, :n].set(x.T)

    degp = _deg_kernel(dst, ones_w, zeros1)
    dis, z1t = _tc1(degp.reshape(N_SC, NPAD), xt, W1.T)
    accp1 = _edge_kernel_4(src, dst, z1t.T, zeros4)
    z2t = _tc2(accp1.transpose(0, 2, 1), z1t, dis, b1.reshape(4, 1), W2.T)
    accp2 = _edge_kernel_8(src, dst, z2t.T, zeros8)
    outt = _tc3(accp2.transpose(0, 2, 1), z2t, dis, b2.reshape(8, 1))
    return outt.T[:n]


# speedup vs baseline: 207.4970x; 1.2510x over previous
import functools

import jax
import jax.numpy as jnp
from jax import lax
from jax.experimental import pallas as pl
from jax.experimental.pallas import tpu as pltpu
from jax.experimental.pallas import tpu_sc as plsc

N_SC = 2
N_TILE = 16
NW = N_SC * N_TILE

N_NODES = 100000
NPAD = 102400
ROWS_PER_TILE = NPAD // N_TILE

N_EDGES = 6400000
EDGES_PER_TILE = N_EDGES // NW

UNROLL = 5
WIN_E = 800
NIT_E = EDGES_PER_TILE // WIN_E
WIN_D = 1600
NIT_D = EDGES_PER_TILE // WIN_D


def _sc_mesh():
    return plsc.VectorSubcoreMesh(core_axis_name="c", subcore_axis_name="s")


def _stage_chunks(window):
    chunks = []
    off = 0
    while off < ROWS_PER_TILE:
        sz = min(window, ROWS_PER_TILE - off)
        chunks.append((off, sz))
        off += sz
    return chunks


@functools.partial(
    pl.kernel,
    out_type=jax.ShapeDtypeStruct((N_SC * NPAD,), jnp.float32),
    mesh=_sc_mesh(),
    scratch_types=[
        [pltpu.VMEM((WIN_D,), jnp.int32) for _ in range(UNROLL)],
        pltpu.VMEM((WIN_D,), jnp.float32),
        pltpu.VMEM_SHARED((NPAD,), jnp.float32),
        [pltpu.SemaphoreType.DMA for _ in range(UNROLL)],
        [pltpu.SemaphoreType.DMA for _ in range(2)],
    ],
)
def _deg_kernel(dst_hbm, ones_hbm, zeros_hbm, out_hbm,
                dst_v, ones_v, acc_sh, sidx, ssc):
    cid = lax.axis_index("c")
    sid = lax.axis_index("s")
    wid = cid * N_TILE + sid
    e0 = wid * EDGES_PER_TILE
    r0 = sid * ROWS_PER_TILE

    pltpu.sync_copy(zeros_hbm.at[pl.ds(0, WIN_D)], ones_v)
    for off, sz in _stage_chunks(WIN_D):
        pltpu.sync_copy(ones_v.at[pl.ds(0, sz)],
                        acc_sh.at[pl.ds(r0 + off, sz)])
    pltpu.sync_copy(ones_hbm, ones_v)
    plsc.subcore_barrier()

    def body(k, carry):
        idx_d = []
        for j in range(UNROLL):
            base = e0 + (k * UNROLL + j) * WIN_D
            idx_d.append(pltpu.async_copy(dst_hbm.at[pl.ds(base, WIN_D)],
                                          dst_v[j], sidx[j]))
        prev = [None, None]
        for j in range(UNROLL):
            idx_d[j].wait()
            if prev[j % 2] is not None:
                prev[j % 2].wait()
            prev[j % 2] = pltpu.async_copy(ones_v, acc_sh.at[dst_v[j]],
                                           ssc[j % 2], add=True)
        for p in prev:
            if p is not None:
                p.wait()
        return carry

    lax.fori_loop(0, NIT_D // UNROLL, body, 0)
    plsc.subcore_barrier()
    for off, sz in _stage_chunks(WIN_D):
        pltpu.sync_copy(acc_sh.at[pl.ds(r0 + off, sz)],
                        ones_v.at[pl.ds(0, sz)])
        pltpu.sync_copy(ones_v.at[pl.ds(0, sz)],
                        out_hbm.at[pl.ds(cid * NPAD + r0 + off, sz)])


def _make_edge_kernel(feat):
    @functools.partial(
        pl.kernel,
        out_type=jax.ShapeDtypeStruct((N_SC, feat, NPAD), jnp.float32),
        mesh=_sc_mesh(),
        scratch_types=[
            [pltpu.VMEM((WIN_E,), jnp.int32) for _ in range(UNROLL)],
            [pltpu.VMEM((WIN_E,), jnp.int32) for _ in range(UNROLL)],
            [pltpu.VMEM((WIN_E, feat), jnp.float32) for _ in range(2)],
            pltpu.VMEM((WIN_E * feat,), jnp.float32),
            pltpu.VMEM_SHARED((NPAD, feat), jnp.float32),
            pltpu.VMEM_SHARED((NPAD, feat), jnp.float32),
            [pltpu.SemaphoreType.DMA for _ in range(UNROLL)],
            pltpu.SemaphoreType.DMA,
            pltpu.SemaphoreType.DMA,
        ],
        compiler_params=pltpu.CompilerParams(use_tc_tiling_on_sc=False,
                                             needs_layout_passes=False),
    )
    def _edge_kernel(src_hbm, dst_hbm, zt_hbm, zeros_hbm, out_hbm,
                     src_v, dst_v, msg_v, col_v, z_sh, acc_sh, sidx, sg, ssc):
        cid = lax.axis_index("c")
        sid = lax.axis_index("s")
        wid = cid * N_TILE + sid
        e0 = wid * EDGES_PER_TILE
        r0 = sid * ROWS_PER_TILE
        lanes = lax.broadcasted_iota(jnp.int32, (16,), 0)
        shift = feat.bit_length() - 1

        pltpu.sync_copy(zeros_hbm, msg_v[0])
        for off, sz in _stage_chunks(WIN_E):
            pltpu.sync_copy(msg_v[0].at[pl.ds(0, sz)],
                            acc_sh.at[pl.ds(r0 + off, sz)])
        for off, sz in _stage_chunks(WIN_E):
            for j in range(feat):
                pltpu.sync_copy(zt_hbm.at[j, pl.ds(r0 + off, sz)],
                                col_v.at[pl.ds(j * WIN_E, sz)])

            def ibody(v, carry):
                b = v * 16
                e = b + lanes
                vals = plsc.load_gather(
                    col_v,
                    [(e & (feat - 1)) * WIN_E + (e >> shift)])
                plsc.store_scatter(msg_v[1], [e >> shift, e & (feat - 1)],
                                   vals)
                return carry

            lax.fori_loop(0, (sz * feat) // 16, ibody, 0)
            pltpu.sync_copy(msg_v[1], z_sh.at[pl.ds(r0 + off, WIN_E)])
        plsc.subcore_barrier()

        def body(k, carry):
            idx_d = []
            for j in range(UNROLL):
                base = e0 + (k * UNROLL + j) * WIN_E
                idx_d.append((
                    pltpu.async_copy(src_hbm.at[pl.ds(base, WIN_E)],
                                     src_v[j], sidx[j]),
                    pltpu.async_copy(dst_hbm.at[pl.ds(base, WIN_E)],
                                     dst_v[j], sidx[j]),
                ))
            prev = None
            for j in range(UNROLL):
                idx_d[j][0].wait()
                idx_d[j][1].wait()
                pltpu.async_copy(z_sh.at[src_v[j]], msg_v[j % 2], sg).wait()
                if prev is not None:
                    prev.wait()
                prev = pltpu.async_copy(msg_v[j % 2], acc_sh.at[dst_v[j]],
                                        ssc, add=True)
            prev.wait()
            return carry

        lax.fori_loop(0, NIT_E // UNROLL, body, 0)
        plsc.subcore_barrier()
        for off, sz in _stage_chunks(WIN_E):
            pltpu.sync_copy(acc_sh.at[pl.ds(r0 + off, sz)], msg_v[0])

            def obody(v, carry):
                b = v * 16
                e = b + lanes
                vals = plsc.load_gather(msg_v[0],
                                        [e >> shift, e & (feat - 1)])
                plsc.store_scatter(
                    col_v,
                    [(e & (feat - 1)) * WIN_E + (e >> shift)], vals)
                return carry

            lax.fori_loop(0, (sz * feat) // 16, obody, 0)
            for j in range(feat):
                pltpu.sync_copy(col_v.at[pl.ds(j * WIN_E, sz)],
                                out_hbm.at[cid, j, pl.ds(r0 + off, sz)])

    return _edge_kernel


_edge_kernel_8 = _make_edge_kernel(8)


def _tc1_body(degp_ref, xt_ref, w1t_ref, dis_ref, z1t_ref):
    deg = degp_ref[0] + degp_ref[1] + 1.0
    dis = lax.rsqrt(deg)[None, :]
    xwt = jnp.dot(w1t_ref[...], xt_ref[...],
                  preferred_element_type=jnp.float32)
    dis_ref[...] = dis
    z1t_ref[0:4, :] = dis * xwt
    z1t_ref[4:8, :] = jnp.zeros((4, NPAD), jnp.float32)


def _tc1(degp, xt, w1t):
    return pl.pallas_call(
        _tc1_body,
        out_shape=(
            jax.ShapeDtypeStruct((1, NPAD), jnp.float32),
            jax.ShapeDtypeStruct((8, NPAD), jnp.float32),
        ),
    )(degp, xt, w1t)


def _tc2_body(accpt_ref, z1t_ref, dis_ref, b1_ref, w2t_ref, z2t_ref):
    dis = dis_ref[...]
    acc = accpt_ref[0, 0:4, :] + accpt_ref[1, 0:4, :]
    agg = dis * (acc + z1t_ref[0:4, :]) + b1_ref[...]
    h = jnp.maximum(agg, 0.0)
    hwt = jnp.dot(w2t_ref[...], h, preferred_element_type=jnp.float32)
    z2t_ref[...] = dis * hwt


def _tc2(accp1t, z1t, dis, b1c, w2t):
    return pl.pallas_call(
        _tc2_body,
        out_shape=jax.ShapeDtypeStruct((8, NPAD), jnp.float32),
    )(accp1t, z1t, dis, b1c, w2t)


def _tc3_body(accpt_ref, z2t_ref, dis_ref, b2_ref, out_ref):
    dis = dis_ref[...]
    out = dis * (accpt_ref[0] + accpt_ref[1] + z2t_ref[...]) + b2_ref[...]
    cols = lax.broadcasted_iota(jnp.int32, (8, NPAD), 1)
    valid = cols < N_NODES
    neg = jnp.full_like(out, -jnp.inf)
    masked = jnp.where(valid, out, neg)
    m = jnp.max(masked, axis=1, keepdims=True)
    s = jnp.sum(jnp.where(valid, jnp.exp(masked - m), 0.0), axis=1,
                keepdims=True)
    out_ref[...] = out - (m + jnp.log(s))


def _tc3(accp2t, z2t, dis, b2c):
    return pl.pallas_call(
        _tc3_body,
        out_shape=jax.ShapeDtypeStruct((8, NPAD), jnp.float32),
    )(accp2t, z2t, dis, b2c)


def kernel(x, edge_index, W1, b1, W2, b2):
    n = x.shape[0]
    src = edge_index[0].astype(jnp.int32)
    dst = edge_index[1].astype(jnp.int32)

    ones_w = jnp.ones((WIN_D,), jnp.float32)
    zeros1 = jnp.zeros((NPAD,), jnp.float32)
    zeros8 = jnp.zeros((WIN_E, 8), jnp.float32)
    xt = jnp.zeros((5, NPAD), jnp.float32).at[:, :n].set(x.T)

    degp = _deg_kernel(dst, ones_w, zeros1)
    dis, z1t = _tc1(degp.reshape(N_SC, NPAD), xt, W1.T)
    accp1t = _edge_kernel_8(src, dst, z1t, zeros8)
    z2t = _tc2(accp1t, z1t, dis, b1.reshape(4, 1), W2.T)
    accp2t = _edge_kernel_8(src, dst, z2t, zeros8)
    outt = _tc3(accp2t, z2t, dis, b2.reshape(8, 1))
    return outt.T[:n]


# speedup vs baseline: 207.6993x; 1.0010x over previous
import functools

import jax
import jax.numpy as jnp
from jax import lax
from jax.experimental import pallas as pl
from jax.experimental.pallas import tpu as pltpu
from jax.experimental.pallas import tpu_sc as plsc

N_SC = 2
N_TILE = 16
NW = N_SC * N_TILE

N_NODES = 100000
NPAD = 102400
ROWS_PER_TILE = NPAD // N_TILE

N_EDGES = 6400000
EDGES_PER_TILE = N_EDGES // NW

UNROLL = 5
WIN_E = 800
NIT_E = EDGES_PER_TILE // WIN_E
WIN_D = 1600
NIT_D = EDGES_PER_TILE // WIN_D


def _sc_mesh():
    return plsc.VectorSubcoreMesh(core_axis_name="c", subcore_axis_name="s")


def _stage_chunks(window):
    chunks = []
    off = 0
    while off < ROWS_PER_TILE:
        sz = min(window, ROWS_PER_TILE - off)
        chunks.append((off, sz))
        off += sz
    return chunks


@functools.partial(
    pl.kernel,
    out_type=jax.ShapeDtypeStruct((N_SC * NPAD,), jnp.float32),
    mesh=_sc_mesh(),
    scratch_types=[
        [pltpu.VMEM((WIN_D,), jnp.int32) for _ in range(UNROLL)],
        pltpu.VMEM((WIN_D,), jnp.float32),
        pltpu.VMEM_SHARED((NPAD,), jnp.float32),
        [pltpu.SemaphoreType.DMA for _ in range(UNROLL)],
        [pltpu.SemaphoreType.DMA for _ in range(2)],
    ],
)
def _deg_kernel(dst_hbm, ones_hbm, zeros_hbm, out_hbm,
                dst_v, ones_v, acc_sh, sidx, ssc):
    cid = lax.axis_index("c")
    sid = lax.axis_index("s")
    wid = cid * N_TILE + sid
    e0 = wid * EDGES_PER_TILE
    r0 = sid * ROWS_PER_TILE

    pltpu.sync_copy(zeros_hbm.at[pl.ds(0, WIN_D)], ones_v)
    for off, sz in _stage_chunks(WIN_D):
        pltpu.sync_copy(ones_v.at[pl.ds(0, sz)],
                        acc_sh.at[pl.ds(r0 + off, sz)])
    pltpu.sync_copy(ones_hbm, ones_v)
    plsc.subcore_barrier()

    def body(k, carry):
        idx_d = []
        for j in range(UNROLL):
            base = e0 + (k * UNROLL + j) * WIN_D
            idx_d.append(pltpu.async_copy(dst_hbm.at[pl.ds(base, WIN_D)],
                                          dst_v[j], sidx[j]))
        prev = [None, None]
        for j in range(UNROLL):
            idx_d[j].wait()
            if prev[j % 2] is not None:
                prev[j % 2].wait()
            prev[j % 2] = pltpu.async_copy(ones_v, acc_sh.at[dst_v[j]],
                                           ssc[j % 2], add=True)
        for p in prev:
            if p is not None:
                p.wait()
        return carry

    lax.fori_loop(0, NIT_D // UNROLL, body, 0)
    plsc.subcore_barrier()
    for off, sz in _stage_chunks(WIN_D):
        pltpu.sync_copy(acc_sh.at[pl.ds(r0 + off, sz)],
                        ones_v.at[pl.ds(0, sz)])
        pltpu.sync_copy(ones_v.at[pl.ds(0, sz)],
                        out_hbm.at[pl.ds(cid * NPAD + r0 + off, sz)])


def _make_edge_kernel(feat):
    @functools.partial(
        pl.kernel,
        out_type=jax.ShapeDtypeStruct((N_SC, feat, NPAD), jnp.float32),
        mesh=_sc_mesh(),
        scratch_types=[
            [pltpu.VMEM((WIN_E,), jnp.int32) for _ in range(UNROLL)],
            [pltpu.VMEM((WIN_E,), jnp.int32) for _ in range(UNROLL)],
            [pltpu.VMEM((WIN_E, feat), jnp.float32) for _ in range(2)],
            pltpu.VMEM((WIN_E * feat,), jnp.float32),
            pltpu.VMEM_SHARED((NPAD, feat), jnp.float32),
            pltpu.VMEM_SHARED((NPAD, feat), jnp.float32),
            [pltpu.SemaphoreType.DMA for _ in range(UNROLL)],
            pltpu.SemaphoreType.DMA,
            pltpu.SemaphoreType.DMA,
        ],
        compiler_params=pltpu.CompilerParams(use_tc_tiling_on_sc=False,
                                             needs_layout_passes=False),
    )
    def _edge_kernel(src_hbm, dst_hbm, zt_hbm, zeros_hbm, out_hbm,
                     src_v, dst_v, msg_v, col_v, z_sh, acc_sh, sidx, sg, ssc):
        cid = lax.axis_index("c")
        sid = lax.axis_index("s")
        wid = cid * N_TILE + sid
        e0 = wid * EDGES_PER_TILE
        r0 = sid * ROWS_PER_TILE
        lanes = lax.broadcasted_iota(jnp.int32, (16,), 0)
        shift = feat.bit_length() - 1

        pltpu.sync_copy(zeros_hbm, msg_v[0])
        for off, sz in _stage_chunks(WIN_E):
            pltpu.sync_copy(msg_v[0].at[pl.ds(0, sz)],
                            acc_sh.at[pl.ds(r0 + off, sz)])
        for off, sz in _stage_chunks(WIN_E):
            for j in range(feat):
                pltpu.sync_copy(zt_hbm.at[j, pl.ds(r0 + off, sz)],
                                col_v.at[pl.ds(j * WIN_E, sz)])

            def ibody(v, carry):
                b = v * 16
                e = b + lanes
                vals = plsc.load_gather(
                    col_v,
                    [(e & (feat - 1)) * WIN_E + (e >> shift)])
                plsc.store_scatter(msg_v[1], [e >> shift, e & (feat - 1)],
                                   vals)
                return carry

            lax.fori_loop(0, (sz * feat) // 16, ibody, 0)
            pltpu.sync_copy(msg_v[1], z_sh.at[pl.ds(r0 + off, WIN_E)])
        plsc.subcore_barrier()

        def body(k, carry):
            idx_d = []
            for j in range(UNROLL):
                base = e0 + (k * UNROLL + j) * WIN_E
                idx_d.append((
                    pltpu.async_copy(src_hbm.at[pl.ds(base, WIN_E)],
                                     src_v[j], sidx[j]),
                    pltpu.async_copy(dst_hbm.at[pl.ds(base, WIN_E)],
                                     dst_v[j], sidx[j]),
                ))
            prev = None
            for j in range(UNROLL):
                idx_d[j][0].wait()
                idx_d[j][1].wait()
                pltpu.async_copy(z_sh.at[src_v[j]], msg_v[j % 2], sg).wait()
                if prev is not None:
                    prev.wait()
                prev = pltpu.async_copy(msg_v[j % 2], acc_sh.at[dst_v[j]],
                                        ssc, add=True)
            prev.wait()
            return carry

        lax.fori_loop(0, NIT_E // UNROLL, body, 0)
        plsc.subcore_barrier()
        for off, sz in _stage_chunks(WIN_E):
            pltpu.sync_copy(acc_sh.at[pl.ds(r0 + off, sz)], msg_v[0])

            def obody(v, carry):
                b = v * 16
                e = b + lanes
                vals = plsc.load_gather(msg_v[0],
                                        [e >> shift, e & (feat - 1)])
                plsc.store_scatter(
                    col_v,
                    [(e & (feat - 1)) * WIN_E + (e >> shift)], vals)
                return carry

            lax.fori_loop(0, (sz * feat) // 16, obody, 0)
            for j in range(feat):
                pltpu.sync_copy(col_v.at[pl.ds(j * WIN_E, sz)],
                                out_hbm.at[cid, j, pl.ds(r0 + off, sz)])

    return _edge_kernel


_edge_kernel_8 = _make_edge_kernel(8)


def _tc1_body(degp_ref, xt_ref, w1t_ref, dis_ref, z1t_ref):
    deg = degp_ref[0] + degp_ref[1] + 1.0
    dis = lax.rsqrt(deg)[None, :]
    xwt = jnp.dot(w1t_ref[...], xt_ref[...],
                  preferred_element_type=jnp.float32)
    dis_ref[...] = dis
    z1t_ref[0:4, :] = dis * xwt
    z1t_ref[4:8, :] = jnp.zeros((4, NPAD), jnp.float32)


def _tc1(degp, xt, w1t):
    return pl.pallas_call(
        _tc1_body,
        out_shape=(
            jax.ShapeDtypeStruct((1, NPAD), jnp.float32),
            jax.ShapeDtypeStruct((8, NPAD), jnp.float32),
        ),
    )(degp, xt, w1t)


def _tc2_body(accpt_ref, z1t_ref, dis_ref, b1_ref, w2t_ref, z2t_ref):
    dis = dis_ref[...]
    acc = accpt_ref[0, 0:4, :] + accpt_ref[1, 0:4, :]
    agg = dis * (acc + z1t_ref[0:4, :]) + b1_ref[...]
    h = jnp.maximum(agg, 0.0)
    hwt = jnp.dot(w2t_ref[...], h, preferred_element_type=jnp.float32)
    z2t_ref[...] = dis * hwt


def _tc2(accp1t, z1t, dis, b1c, w2t):
    return pl.pallas_call(
        _tc2_body,
        out_shape=jax.ShapeDtypeStruct((8, NPAD), jnp.float32),
    )(accp1t, z1t, dis, b1c, w2t)


def _tc3_body(accpt_ref, z2t_ref, dis_ref, b2_ref, out_ref):
    dis = dis_ref[...]
    out = dis * (accpt_ref[0] + accpt_ref[1] + z2t_ref[...]) + b2_ref[...]
    cols = lax.broadcasted_iota(jnp.int32, (8, NPAD), 1)
    valid = cols < N_NODES
    neg = jnp.full_like(out, -jnp.inf)
    masked = jnp.where(valid, out, neg)
    m = jnp.max(masked, axis=1, keepdims=True)
    s = jnp.sum(jnp.where(valid, jnp.exp(masked - m), 0.0), axis=1,
                keepdims=True)
    out_ref[...] = out - (m + jnp.log(s))


def _tc3(accp2t, z2t, dis, b2c):
    return pl.pallas_call(
        _tc3_body,
        out_shape=jax.ShapeDtypeStruct((8, NPAD), jnp.float32),
    )(accp2t, z2t, dis, b2c)


def kernel(x, edge_index, W1, b1, W2, b2):
    n = x.shape[0]
    src = edge_index[0].astype(jnp.int32)
    dst = edge_index[1].astype(jnp.int32)

    ones_w = jnp.ones((WIN_D,), jnp.float32)
    zeros1 = jnp.zeros((NPAD,), jnp.float32)
    zeros8 = jnp.zeros((WIN_E, 8), jnp.float32)

    degp = _deg_kernel(dst, ones_w, zeros1)
    xt = jnp.zeros((5, NPAD), jnp.float32).at[:, :n].set(x.T)
    dis, z1t = _tc1(degp.reshape(N_SC, NPAD), xt, W1.T)
    accp1t = _edge_kernel_8(src, dst, z1t, zeros8)
    z2t = _tc2(accp1t, z1t, dis, b1.reshape(4, 1), W2.T)
    accp2t = _edge_kernel_8(src, dst, z2t, zeros8)
    outt = _tc3(accp2t, z2t, dis, b2.reshape(8, 1))
    return outt.T[:n]
